# baseline (speedup 1.0000x reference)
import functools

import jax
import jax.numpy as jnp
from jax.experimental import pallas as pl
from jax.experimental.pallas import tpu as pltpu


def _fused_kernel(sx_ref, nx_ref, adj_t_ref, adj_b_ref, sw_ref, nw_ref,
                  sb_ref, nb_ref, out_ref):
    f32 = jnp.float32
    nx = nx_ref[...]
    nw = nw_ref[...]
    sw = sw_ref[...]
    for half, adj_ref in ((0, adj_t_ref), (1, adj_b_ref)):
        agg = jnp.dot(adj_ref[...], nx, preferred_element_type=f32)
        nbr = jnp.dot(agg, nw, preferred_element_type=f32) + nb_ref[...]
        slf = jnp.dot(sx_ref[half], sw, preferred_element_type=f32) + sb_ref[...]
        out_ref[half] = jnp.maximum(jnp.concatenate([slf, nbr], axis=1), 0.0)


@functools.partial(jax.jit, static_argnames=("bm",))
def _run(self_x, neighbor_x, adj, self_weight, neighbor_weight,
         self_bias, neighbor_bias, bm):
    n, d_in = self_x.shape
    d_out = self_weight.shape[1]
    half_blocks = (n // 2) // bm
    grid = (half_blocks,)
    sx3 = self_x.reshape(2, n // 2, d_in)
    out = pl.pallas_call(
        _fused_kernel,
        grid=grid,
        in_specs=[
            pl.BlockSpec((2, bm, d_in), lambda i: (0, i, 0)),
            pl.BlockSpec((n, d_in), lambda i: (0, 0)),
            pl.BlockSpec((bm, n), lambda i: (i, 0)),
            pl.BlockSpec((bm, n), lambda i, hb=half_blocks: (i + hb, 0)),
            pl.BlockSpec((d_in, d_out), lambda i: (0, 0)),
            pl.BlockSpec((d_in, d_out), lambda i: (0, 0)),
            pl.BlockSpec((1, d_out), lambda i: (0, 0)),
            pl.BlockSpec((1, d_out), lambda i: (0, 0)),
        ],
        out_specs=pl.BlockSpec((2, bm, 2 * d_out), lambda i: (0, i, 0)),
        out_shape=jax.ShapeDtypeStruct((2, n // 2, 2 * d_out), jnp.float32),
        compiler_params=pltpu.CompilerParams(
            dimension_semantics=("arbitrary",),
        ),
    )(sx3, neighbor_x, adj, adj, self_weight, neighbor_weight,
      self_bias, neighbor_bias)
    return out.reshape(n, 2 * d_out)


def kernel(self_x, neighbor_x, adj, self_weight, neighbor_weight,
           self_bias, neighbor_bias):
    n = adj.shape[0]
    bm = next(b for b in (200, 100, 8, 1) if (n // 2) % b == 0)
    sb = self_bias.reshape(1, -1)
    nb = neighbor_bias.reshape(1, -1)
    return _run(self_x, neighbor_x, adj, self_weight, neighbor_weight,
                sb, nb, bm)


# speedup vs baseline: 1.0862x; 1.0862x over previous
import functools

import jax
import jax.numpy as jnp
from jax.experimental import pallas as pl
from jax.experimental.pallas import tpu as pltpu


def _fused_kernel(sx_ref, nx_ref, adj_ref, sw_ref, nw_ref, sb_ref, nb_ref,
                  out_ref):
    f32 = jnp.float32
    agg = jnp.dot(adj_ref[...], nx_ref[...], preferred_element_type=f32)
    nbr = jnp.dot(agg, nw_ref[...], preferred_element_type=f32) + nb_ref[...]
    slf = jnp.dot(sx_ref[...], sw_ref[...], preferred_element_type=f32) + sb_ref[...]
    out_ref[...] = jnp.maximum(jnp.concatenate([slf, nbr], axis=1), 0.0)


@functools.partial(jax.jit, static_argnames=("bm",))
def _run(self_x, neighbor_x, adj, self_weight, neighbor_weight,
         self_bias, neighbor_bias, bm):
    n, d_in = self_x.shape
    d_out = self_weight.shape[1]
    grid = (pl.cdiv(n, bm),)
    return pl.pallas_call(
        _fused_kernel,
        grid=grid,
        in_specs=[
            pl.BlockSpec((bm, d_in), lambda i: (i, 0)),
            pl.BlockSpec((n, d_in), lambda i: (0, 0)),
            pl.BlockSpec((bm, n), lambda i: (i, 0)),
            pl.BlockSpec((d_in, d_out), lambda i: (0, 0)),
            pl.BlockSpec((d_in, d_out), lambda i: (0, 0)),
            pl.BlockSpec((1, d_out), lambda i: (0, 0)),
            pl.BlockSpec((1, d_out), lambda i: (0, 0)),
        ],
        out_specs=pl.BlockSpec((bm, 2 * d_out), lambda i: (i, 0)),
        out_shape=jax.ShapeDtypeStruct((n, 2 * d_out), jnp.float32),
        compiler_params=pltpu.CompilerParams(
            dimension_semantics=("arbitrary",),
        ),
    )(self_x, neighbor_x, adj, self_weight, neighbor_weight,
      self_bias, neighbor_bias)


def kernel(self_x, neighbor_x, adj, self_weight, neighbor_weight,
           self_bias, neighbor_bias):
    sb = self_bias.reshape(1, -1)
    nb = neighbor_bias.reshape(1, -1)
    return _run(self_x, neighbor_x, adj, self_weight, neighbor_weight,
                sb, nb, 512)


# speedup vs baseline: 1.0953x; 1.0084x over previous
import functools

import jax
import jax.numpy as jnp
from jax.experimental import pallas as pl
from jax.experimental.pallas import tpu as pltpu

_NBUF = 4


def _fused_kernel(sx_ref, nx_ref, adj_hbm, sw_ref, nw_ref, sb_ref, nb_ref,
                  out_ref, buf, sems):
    f32 = jnp.float32
    i = pl.program_id(0)
    nsteps = pl.num_programs(0)
    bm = out_ref.shape[0]

    def issue(step):
        slot = jax.lax.rem(step, _NBUF)
        pltpu.make_async_copy(
            adj_hbm.at[pl.ds(step * bm, bm), :],
            buf.at[slot],
            sems.at[slot],
        ).start()

    @pl.when(i == 0)
    def _warmup():
        for j in range(_NBUF - 1):
            issue(jnp.int32(j))

    nxt = i + _NBUF - 1

    @pl.when(nxt < nsteps)
    def _prefetch():
        issue(nxt)

    slot = jax.lax.rem(i, _NBUF)
    pltpu.make_async_copy(
        adj_hbm.at[pl.ds(i * bm, bm), :],
        buf.at[slot],
        sems.at[slot],
    ).wait()

    agg = jnp.dot(buf[slot], nx_ref[...], preferred_element_type=f32)
    nbr = jnp.dot(agg, nw_ref[...], preferred_element_type=f32) + nb_ref[...]
    slf = jnp.dot(sx_ref[...], sw_ref[...], preferred_element_type=f32) + sb_ref[...]
    out_ref[...] = jnp.maximum(jnp.concatenate([slf, nbr], axis=1), 0.0)


@functools.partial(jax.jit, static_argnames=("bm",))
def _run(self_x, neighbor_x, adj, self_weight, neighbor_weight,
         self_bias, neighbor_bias, bm):
    n, d_in = self_x.shape
    d_out = self_weight.shape[1]
    grid = (n // bm,)
    return pl.pallas_call(
        _fused_kernel,
        grid=grid,
        in_specs=[
            pl.BlockSpec((bm, d_in), lambda i: (i, 0)),
            pl.BlockSpec((n, d_in), lambda i: (0, 0)),
            pl.BlockSpec(memory_space=pl.ANY),
            pl.BlockSpec((d_in, d_out), lambda i: (0, 0)),
            pl.BlockSpec((d_in, d_out), lambda i: (0, 0)),
            pl.BlockSpec((1, d_out), lambda i: (0, 0)),
            pl.BlockSpec((1, d_out), lambda i: (0, 0)),
        ],
        out_specs=pl.BlockSpec((bm, 2 * d_out), lambda i: (i, 0)),
        out_shape=jax.ShapeDtypeStruct((n, 2 * d_out), jnp.float32),
        scratch_shapes=[
            pltpu.VMEM((_NBUF, bm, n), jnp.float32),
            pltpu.SemaphoreType.DMA((_NBUF,)),
        ],
        compiler_params=pltpu.CompilerParams(
            dimension_semantics=("arbitrary",),
        ),
    )(self_x, neighbor_x, adj, self_weight, neighbor_weight,
      self_bias, neighbor_bias)


def kernel(self_x, neighbor_x, adj, self_weight, neighbor_weight,
           self_bias, neighbor_bias):
    n = adj.shape[0]
    bm = next(b for b in (200, 100, 8, 1) if n % b == 0)
    sb = self_bias.reshape(1, -1)
    nb = neighbor_bias.reshape(1, -1)
    return _run(self_x, neighbor_x, adj, self_weight, neighbor_weight,
                sb, nb, bm)


# speedup vs baseline: 1.0967x; 1.0013x over previous
import functools

import jax
import jax.numpy as jnp
from jax.experimental import pallas as pl
from jax.experimental.pallas import tpu as pltpu


def _fused_kernel(sx_ref, nx_ref, adj_ref, sw_ref, nw_ref, sb_ref, nb_ref,
                  out_ref):
    f32 = jnp.float32
    agg = jnp.dot(adj_ref[...], nx_ref[...], preferred_element_type=f32)
    nbr = jnp.dot(agg, nw_ref[...], preferred_element_type=f32) + nb_ref[...]
    slf = jnp.dot(sx_ref[...], sw_ref[...], preferred_element_type=f32) + sb_ref[...]
    out_ref[...] = jnp.maximum(jnp.concatenate([slf, nbr], axis=1), 0.0)


@functools.partial(jax.jit, static_argnames=("bm",))
def _run(self_x, neighbor_x, adj, self_weight, neighbor_weight,
         self_bias, neighbor_bias, bm):
    n, d_in = self_x.shape
    d_out = self_weight.shape[1]
    grid = (n // bm,)
    return pl.pallas_call(
        _fused_kernel,
        grid=grid,
        in_specs=[
            pl.BlockSpec((bm, d_in), lambda i: (i, 0)),
            pl.BlockSpec((n, d_in), lambda i: (0, 0)),
            pl.BlockSpec((bm, n), lambda i: (i, 0)),
            pl.BlockSpec((d_in, d_out), lambda i: (0, 0)),
            pl.BlockSpec((d_in, d_out), lambda i: (0, 0)),
            pl.BlockSpec((1, d_out), lambda i: (0, 0)),
            pl.BlockSpec((1, d_out), lambda i: (0, 0)),
        ],
        out_specs=pl.BlockSpec((bm, 2 * d_out), lambda i: (i, 0)),
        out_shape=jax.ShapeDtypeStruct((n, 2 * d_out), jnp.float32),
        compiler_params=pltpu.CompilerParams(
            dimension_semantics=("arbitrary",),
        ),
    )(self_x, neighbor_x, adj, self_weight, neighbor_weight,
      self_bias, neighbor_bias)


def kernel(self_x, neighbor_x, adj, self_weight, neighbor_weight,
           self_bias, neighbor_bias):
    n = adj.shape[0]
    bm = next(b for b in (400, 200, 100, 8, 1) if n % b == 0)
    sb = self_bias.reshape(1, -1)
    nb = neighbor_bias.reshape(1, -1)
    return _run(self_x, neighbor_x, adj, self_weight, neighbor_weight,
                sb, nb, bm)
